# baseline (speedup 1.0000x reference)
import functools
import math

import jax
import jax.numpy as jnp
import numpy as np
from jax import lax
from jax.experimental import pallas as pl
from jax.experimental.pallas import tpu as pltpu
from jax.experimental.pallas import tpu_sc as plsc

N0 = 10000
NPAD = 10240
E = 320000
D = 128
L = 10
RATIO = 0.8

NC = 2
NS = 16
NW = NC * NS
CHUNK = 128
NCHUNKS = E // CHUNK
CH_PER_W = -(-NCHUNKS // NW)
STRIPE = NPAD // NS


def _sc_deg_build():
    mesh = plsc.VectorSubcoreMesh(core_axis_name="c", subcore_axis_name="s")

    def body(vals_hbm, src_hbm, dst_hbm, zeros_hbm, out_hbm,
             src_v, dst_v, rows_v, acc_sh, gsem):
        cid_core = lax.axis_index("c")
        sid = lax.axis_index("s")
        wid = sid * NC + cid_core

        pltpu.sync_copy(zeros_hbm.at[pl.ds(sid * STRIPE, STRIPE)],
                        acc_sh.at[pl.ds(sid * STRIPE, STRIPE)])
        plsc.subcore_barrier()

        def step(j, carry):
            cid = wid + j * NW

            @pl.when(cid < NCHUNKS)
            def _():
                pltpu.sync_copy(src_hbm.at[pl.ds(cid * CHUNK, CHUNK)], src_v)
                pltpu.sync_copy(dst_hbm.at[pl.ds(cid * CHUNK, CHUNK)], dst_v)
                pltpu.async_copy(vals_hbm.at[src_v], rows_v, gsem).wait()
                pltpu.sync_copy(rows_v, acc_sh.at[dst_v], add=True)

            return carry

        lax.fori_loop(0, CH_PER_W, step, 0)
        plsc.subcore_barrier()
        pltpu.sync_copy(
            acc_sh.at[pl.ds(sid * STRIPE, STRIPE)],
            out_hbm.at[pl.ds(cid_core * NPAD + sid * STRIPE, STRIPE)])

    return pl.kernel(
        body,
        out_type=jax.ShapeDtypeStruct((NC * NPAD, D), jnp.float32),
        mesh=mesh,
        scratch_types=[
            pltpu.VMEM((CHUNK,), jnp.int32),
            pltpu.VMEM((CHUNK,), jnp.int32),
            pltpu.VMEM((CHUNK, D), jnp.float32),
            pltpu.VMEM_SHARED((NPAD, D), jnp.float32),
            pltpu.SemaphoreType.DMA,
        ],
    )


_sc_deg = _sc_deg_build()


def _dotb(x, w):
    return jnp.dot(x.astype(jnp.bfloat16), w.astype(jnp.bfloat16),
                   preferred_element_type=jnp.float32)


def _mm2_body(x_ref, w_ref, v_ref, h_ref, xv_ref):
    x = x_ref[...]
    h_ref[...] = _dotb(x, w_ref[...])
    xv_ref[...] = _dotb(x, v_ref[...])


_mm2 = pl.pallas_call(
    _mm2_body,
    out_shape=[
        jax.ShapeDtypeStruct((NPAD, D), jnp.float32),
        jax.ShapeDtypeStruct((NPAD, D), jnp.float32),
    ],
)


def _bn_body(n, t_ref, mean_ref, var_ref, g_ref, be_ref, a_ref, y_ref):
    t = t_ref[...]
    y = g_ref[...] * (t - mean_ref[...]) / jnp.sqrt(var_ref[...] + 1e-5) \
        + be_ref[...]
    y = jnp.where(y >= 0, y, a_ref[0, 0] * y)
    rows = lax.broadcasted_iota(jnp.int32, (NPAD, 1), 0)
    y_ref[...] = jnp.where(rows < n, y, 0.0)


@functools.cache
def _bn(n):
    return pl.pallas_call(
        functools.partial(_bn_body, n),
        out_shape=jax.ShapeDtypeStruct((NPAD, D), jnp.float32),
    )


def _t_body(n, agg_ref, xv_ref, b_ref, t_ref):
    t = jnp.maximum(agg_ref[...] + xv_ref[...] + b_ref[...], 0.0)
    rows = lax.broadcasted_iota(jnp.int32, (NPAD, 1), 0)
    t_ref[...] = jnp.where(rows < n, t, 0.0)


@functools.cache
def _tk(n):
    return pl.pallas_call(
        functools.partial(_t_body, n),
        out_shape=jax.ShapeDtypeStruct((NPAD, D), jnp.float32),
    )


def _reads_body(k, x_ref, reads_ref):
    xn = x_ref[...]
    rows = lax.broadcasted_iota(jnp.int32, (NPAD, 1), 0)
    live = rows < k
    reads_ref[0:1, :] = jnp.max(jnp.where(live, xn, -3.4e38), axis=0,
                                keepdims=True)
    reads_ref[1:2, :] = jnp.sum(jnp.where(live, xn, 0.0), axis=0,
                                keepdims=True) / k


@functools.cache
def _reads(k):
    return pl.pallas_call(
        functools.partial(_reads_body, k),
        out_shape=jax.ShapeDtypeStruct((2, D), jnp.float32),
    )


def _tc3_body(xc_ref, w1_ref, b1_ref, w2_ref, b2_ref, a_ref, out_ref):
    a = a_ref[0, 0]
    h1 = _dotb(xc_ref[...], w1_ref[...]) + b1_ref[...]
    h1 = jnp.where(h1 >= 0, h1, a * h1)
    o = _dotb(h1, w2_ref[...]) + b2_ref[...]
    o = jnp.where(o >= 0, o, a * o)
    o = o - jnp.min(o, axis=1, keepdims=True)
    o = o / jnp.max(o, axis=1, keepdims=True)
    o = o / jnp.sum(o, axis=1, keepdims=True)
    out_ref[...] = o


_tc3 = pl.pallas_call(
    _tc3_body,
    out_shape=jax.ShapeDtypeStruct((8, 10), jnp.float32),
)


def kernel(x, edge_index, batch, Ws, Vs, bs, gammas, betas, ps,
           lin1_W, lin1_b, lin2_W, lin2_b, prelu_a):
    src0 = edge_index[0].astype(jnp.int32)
    dst0 = edge_index[1].astype(jnp.int32)
    src_c, dst_c = src0, dst0
    emask = jnp.ones(E, jnp.float32)
    ids = jnp.arange(N0, dtype=jnp.int32)
    xp = jnp.zeros((NPAD, D), jnp.float32).at[:N0].set(x)
    zerosD = jnp.zeros((NPAD, D), jnp.float32)
    a11 = prelu_a.reshape(1, 1)

    n = N0
    reads = []
    for i in range(L):
        k = int(math.ceil(RATIO * n))
        m128 = jnp.zeros((NPAD, D), jnp.float32).at[ids].set(1.0)
        degp = _sc_deg(m128, src0, dst0, zerosD)
        deg = (degp[:NPAD, 0] + degp[NPAD:, 0])[ids]
        dinv = jnp.where(deg > 0, 1.0 / jnp.sqrt(deg), 0.0)
        norm = dinv[src_c] * dinv[dst_c] * emask
        hp, xvp = _mm2(xp, Ws[i], Vs[i])
        h = hp[:n]
        agg = jax.ops.segment_sum(h[src_c] * norm[:, None], dst_c,
                                  num_segments=n)
        aggp = jnp.zeros((NPAD, D), jnp.float32).at[:n].set(agg)
        tp = _tk(n)(aggp, xvp, bs[i].reshape(1, D))
        t = tp[:n]
        mean = t.mean(axis=0)
        var = t.var(axis=0)
        yp = _bn(n)(tp, mean.reshape(1, D), var.reshape(1, D),
                    gammas[i].reshape(1, D), betas[i].reshape(1, D), a11)
        y = yp[:n]
        p = ps[i]
        score = jnp.tanh((y @ p) / jnp.linalg.norm(p))
        vals, perm = jax.lax.top_k(score, k)
        xn = y[perm] * vals[:, None]
        sel = jnp.zeros(n, jnp.float32).at[perm].set(1.0)
        new_idx = jnp.zeros(n, jnp.int32).at[perm].set(
            jnp.arange(k, dtype=jnp.int32))
        emask = emask * sel[src_c] * sel[dst_c]
        src_c = new_idx[src_c]
        dst_c = new_idx[dst_c]
        ids = ids[perm]
        xp = jnp.zeros((NPAD, D), jnp.float32).at[:k].set(xn)
        n = k
        reads.append(_reads(k)(xp))

    xc = jnp.stack(reads).reshape(1, 2 * D * L)
    xc8 = jnp.zeros((8, 2 * D * L), jnp.float32).at[0:1].set(xc)
    out8 = _tc3(xc8, lin1_W, lin1_b.reshape(1, -1), lin2_W,
                lin2_b.reshape(1, -1), a11)
    return out8[0:1]
